# baseline (speedup 1.0000x reference)
import functools

import jax
import jax.numpy as jnp
from jax import lax
from jax.experimental import pallas as pl
from jax.experimental.pallas import tpu as pltpu

N = 2048
F = 100
C = 100
K = 20
D = 4096
BLK = 128
NB = N // BLK
CH = 512
CG = 8


def _prep_body(fp_ref, ti_ref, cr_ref, ws_ref, cf_ref, ci_ref, pi_ref):
    fp = fp_ref[...]
    ti = ti_ref[...]
    cr = cr_ref[...]

    lane_n = lax.broadcasted_iota(jnp.int32, (CH, N), 1).astype(jnp.float32)
    fpg_chunks = []
    for j in range(N // CH):
        tic = ti[CH * j:CH * (j + 1)]
        oh = (lane_n == tic).astype(jnp.float32)
        fpg_chunks.append(
            jnp.dot(oh, fp, preferred_element_type=jnp.float32,
                    precision=lax.Precision.HIGHEST))
    fpg = jnp.concatenate(fpg_chunks, axis=0)

    m = jnp.max(fpg, axis=1, keepdims=True)
    z = jnp.sum(jnp.exp(fpg - m), axis=1, keepdims=True)
    lane_f = lax.broadcasted_iota(jnp.int32, (N, F), 1).astype(jnp.float32)
    x = fpg
    wd = jnp.zeros((N, F), jnp.float32)
    for _ in range(K):
        vmax = jnp.max(x, axis=1, keepdims=True)
        fidx = jnp.min(jnp.where(x == vmax, lane_f, jnp.float32(1e9)),
                       axis=1, keepdims=True)
        sel = lane_f == fidx
        w = jnp.exp(vmax - m) / (z * 255.0)
        wd = wd + jnp.where(sel, w, 0.0)
        x = jnp.where(sel, jnp.float32(-1e30), x)

    cm = jnp.max(cr, axis=1, keepdims=True)
    cidx = jnp.min(jnp.where(cr == cm, lane_f, jnp.float32(1e9)),
                   axis=1, keepdims=True)

    onehot_c = (lane_f == cidx).astype(jnp.float32)
    cum = onehot_c
    s = 1
    while s < N:
        cum = cum + jnp.concatenate(
            [jnp.zeros((s, C), jnp.float32), cum[:N - s]], axis=0)
        s *= 2
    counts = cum[N - 1:N]
    inc = counts
    s = 1
    while s < C:
        inc = inc + jnp.concatenate(
            [jnp.zeros((1, s), jnp.float32), inc[:, :C - s]], axis=1)
        s *= 2
    offs = inc - counts
    rank = jnp.sum(onehot_c * cum, axis=1, keepdims=True) - 1.0
    base = jnp.sum(onehot_c * offs, axis=1, keepdims=True)
    pos = base + rank

    lane_ch = lax.broadcasted_iota(jnp.int32, (N, CH), 1).astype(jnp.float32)
    n_iota = lax.broadcasted_iota(jnp.int32, (N, 1), 0).astype(jnp.float32)
    dn = (((0,), (0,)), ((), ()))
    for j in range(N // CH):
        t = (pos == lane_ch + jnp.float32(CH * j)).astype(jnp.float32)
        ws_ref[CH * j:CH * (j + 1), :] = lax.dot_general(
            t, wd, dn, preferred_element_type=jnp.float32,
            precision=lax.Precision.HIGHEST)
        cf = lax.dot_general(t, cidx, dn, preferred_element_type=jnp.float32,
                             precision=lax.Precision.HIGHEST)
        pf = lax.dot_general(t, n_iota, dn, preferred_element_type=jnp.float32,
                             precision=lax.Precision.HIGHEST)
        cf_ref[CH * j:CH * (j + 1), :] = cf
        ci_ref[CH * j:CH * (j + 1), :] = cf.astype(jnp.int32)
        pi_ref[CH * j:CH * (j + 1), :] = pf.astype(jnp.int32)


def _combine_body(ci_sm, pi_sm, w_ref, cvec_ref, atlas_ref, out_ref,
                  g0, g1, acc, sem_g, sem_out):
    i = pl.program_id(0)
    row0 = i * BLK
    c_lo = ci_sm[row0]
    c_hi = ci_sm[row0 + BLK - 1]
    num = c_hi - c_lo + 1
    nt = (num + CG - 1) // CG

    def start_of(t):
        return jnp.minimum(c_lo + CG * t, C - CG)

    def issue(t):
        off = pl.multiple_of(start_of(t) * D, 128)
        src_slab = atlas_ref.at[:, pl.ds(off, CG * D)]

        @pl.when(lax.rem(t, 2) == 0)
        def _():
            pltpu.make_async_copy(src_slab, g0, sem_g.at[0]).start()

        @pl.when(lax.rem(t, 2) == 1)
        def _():
            pltpu.make_async_copy(src_slab, g1, sem_g.at[1]).start()

    issue(0)
    acc[...] = jnp.zeros((BLK, D), jnp.float32)
    cvec = cvec_ref[...]
    w = w_ref[...]

    def consume(t, gref, sem):
        off = pl.multiple_of(start_of(t) * D, 128)
        pltpu.make_async_copy(atlas_ref.at[:, pl.ds(off, CG * D)],
                              gref, sem).wait()
        st = start_of(t)
        for j in range(CG):
            cs = st + j

            @pl.when((cs >= c_lo + CG * t) & (cs <= c_hi))
            def _():
                mask = (cvec == cs.astype(jnp.float32)).astype(jnp.float32)
                acc[...] = acc[...] + jnp.dot(
                    w * mask, gref[:, j * D:(j + 1) * D],
                    preferred_element_type=jnp.float32)

    def body(t, carry):
        @pl.when(t + 1 < nt)
        def _():
            issue(t + 1)

        @pl.when(lax.rem(t, 2) == 0)
        def _():
            consume(t, g0, sem_g.at[0])

        @pl.when(lax.rem(t, 2) == 1)
        def _():
            consume(t, g1, sem_g.at[1])

        return carry

    lax.fori_loop(0, nt, body, 0)

    dmas = []
    for r in range(BLK):
        d = pltpu.make_async_copy(acc.at[r], out_ref.at[pi_sm[row0 + r]],
                                  sem_out)
        d.start()
        dmas.append(d)
    for d in dmas:
        d.wait()


def _prep_call(fp2d, ti_f32, cr):
    return pl.pallas_call(
        _prep_body,
        out_shape=(
            jax.ShapeDtypeStruct((N, F), jnp.float32),
            jax.ShapeDtypeStruct((N, 1), jnp.float32),
            jax.ShapeDtypeStruct((N, 1), jnp.int32),
            jax.ShapeDtypeStruct((N, 1), jnp.int32),
        ),
    )(fp2d, ti_f32, cr)


def _combine_call(ci, pi, ws, cf, atlas3):
    grid_spec = pltpu.PrefetchScalarGridSpec(
        num_scalar_prefetch=2,
        grid=(NB,),
        in_specs=[
            pl.BlockSpec((BLK, F), lambda i, *_: (i, 0)),
            pl.BlockSpec((BLK, 1), lambda i, *_: (i, 0)),
            pl.BlockSpec(memory_space=pltpu.MemorySpace.HBM),
        ],
        out_specs=pl.BlockSpec(memory_space=pltpu.MemorySpace.HBM),
        scratch_shapes=[
            pltpu.VMEM((F, CG * D), jnp.float32),
            pltpu.VMEM((F, CG * D), jnp.float32),
            pltpu.VMEM((BLK, D), jnp.float32),
            pltpu.SemaphoreType.DMA((2,)),
            pltpu.SemaphoreType.DMA,
        ],
    )
    return pl.pallas_call(
        _combine_body,
        grid_spec=grid_spec,
        out_shape=jax.ShapeDtypeStruct((N, D), jnp.float32),
    )(ci, pi, ws, cf, atlas3)


@jax.jit
def kernel(font_pred, char_labels, char_rec_vec, text_indexes,
           prerendered_alpha):
    del char_labels
    b, t, f = font_pred.shape
    fp2d = font_pred.reshape(b * t, f)
    ti_f32 = text_indexes.reshape(N, 1).astype(jnp.float32)
    ws, cf, ci, pi = _prep_call(fp2d, ti_f32, char_rec_vec)
    atlas3 = prerendered_alpha.reshape(F, C * D)
    out = _combine_call(ci.reshape(N), pi.reshape(N), ws, cf, atlas3)
    return out.reshape(N, 64, 64)


# speedup vs baseline: 1.0903x; 1.0903x over previous
import functools

import jax
import jax.numpy as jnp
from jax import lax
from jax.experimental import pallas as pl
from jax.experimental.pallas import tpu as pltpu

N = 2048
F = 100
C = 100
K = 20
D = 4096
BLK = 128
NB = N // BLK
CH = 512
RB = 8
LA = 6


def _prep_body(fp_ref, ti_ref, cr_ref, ws_ref, cf_ref, ci_ref, pi_ref):
    fp = fp_ref[...]
    ti = ti_ref[...]
    cr = cr_ref[...]

    lane_n = lax.broadcasted_iota(jnp.int32, (CH, N), 1).astype(jnp.float32)
    fpg_chunks = []
    for j in range(N // CH):
        tic = ti[CH * j:CH * (j + 1)]
        oh = (lane_n == tic).astype(jnp.float32)
        fpg_chunks.append(
            jnp.dot(oh, fp, preferred_element_type=jnp.float32,
                    precision=lax.Precision.HIGHEST))
    fpg = jnp.concatenate(fpg_chunks, axis=0)

    m = jnp.max(fpg, axis=1, keepdims=True)
    z = jnp.sum(jnp.exp(fpg - m), axis=1, keepdims=True)
    lane_f = lax.broadcasted_iota(jnp.int32, (N, F), 1).astype(jnp.float32)
    x = fpg
    wd = jnp.zeros((N, F), jnp.float32)
    for _ in range(K):
        vmax = jnp.max(x, axis=1, keepdims=True)
        fidx = jnp.min(jnp.where(x == vmax, lane_f, jnp.float32(1e9)),
                       axis=1, keepdims=True)
        sel = lane_f == fidx
        w = jnp.exp(vmax - m) / (z * 255.0)
        wd = wd + jnp.where(sel, w, 0.0)
        x = jnp.where(sel, jnp.float32(-1e30), x)

    cm = jnp.max(cr, axis=1, keepdims=True)
    cidx = jnp.min(jnp.where(cr == cm, lane_f, jnp.float32(1e9)),
                   axis=1, keepdims=True)

    onehot_c = (lane_f == cidx).astype(jnp.float32)
    cum = onehot_c
    s = 1
    while s < N:
        cum = cum + jnp.concatenate(
            [jnp.zeros((s, C), jnp.float32), cum[:N - s]], axis=0)
        s *= 2
    counts = cum[N - 1:N]
    inc = counts
    s = 1
    while s < C:
        inc = inc + jnp.concatenate(
            [jnp.zeros((1, s), jnp.float32), inc[:, :C - s]], axis=1)
        s *= 2
    offs = inc - counts
    rank = jnp.sum(onehot_c * cum, axis=1, keepdims=True) - 1.0
    base = jnp.sum(onehot_c * offs, axis=1, keepdims=True)
    pos = base + rank

    lane_ch = lax.broadcasted_iota(jnp.int32, (N, CH), 1).astype(jnp.float32)
    n_iota = lax.broadcasted_iota(jnp.int32, (N, 1), 0).astype(jnp.float32)
    dn = (((0,), (0,)), ((), ()))
    for j in range(N // CH):
        t = (pos == lane_ch + jnp.float32(CH * j)).astype(jnp.float32)
        ws_ref[CH * j:CH * (j + 1), :] = lax.dot_general(
            t, wd, dn, preferred_element_type=jnp.float32,
            precision=lax.Precision.HIGHEST)
        cf = lax.dot_general(t, cidx, dn, preferred_element_type=jnp.float32,
                             precision=lax.Precision.HIGHEST)
        pf = lax.dot_general(t, n_iota, dn, preferred_element_type=jnp.float32,
                             precision=lax.Precision.HIGHEST)
        cf_ref[CH * j:CH * (j + 1), :] = cf
        ci_ref[CH * j:CH * (j + 1), :] = cf.astype(jnp.int32)
        pi_ref[CH * j:CH * (j + 1), :] = pf.astype(jnp.int32)


def _combine_body(ci_sm, pi_sm, w_ref, cvec_ref, atlas_ref, out_ref,
                  g_buf, acc, sem_g, sem_out):
    i = pl.program_id(0)
    row0 = i * BLK
    c_lo = ci_sm[row0]
    c_hi = ci_sm[row0 + BLK - 1]
    num = c_hi - c_lo + 1

    def slab(t):
        off = pl.multiple_of((c_lo + t) * D, 128)
        return atlas_ref.at[:, pl.ds(off, D)]

    def issue(t):
        slot = lax.rem(t, RB)
        pltpu.make_async_copy(slab(t), g_buf.at[slot], sem_g.at[slot]).start()

    for tt in range(LA):
        @pl.when(tt < num)
        def _(tt=tt):
            issue(tt)

    acc[...] = jnp.zeros((BLK, D), jnp.float32)
    cvec = cvec_ref[...]
    w = w_ref[...]

    def body(t, carry):
        @pl.when(t + LA < num)
        def _():
            issue(t + LA)

        slot = lax.rem(t, RB)
        pltpu.make_async_copy(slab(t), g_buf.at[slot], sem_g.at[slot]).wait()
        mask = (cvec == (c_lo + t).astype(jnp.float32)).astype(jnp.float32)
        acc[...] = acc[...] + jnp.dot(w * mask, g_buf[slot],
                                      preferred_element_type=jnp.float32)
        return carry

    lax.fori_loop(0, num, body, 0)

    dmas = []
    for r in range(BLK):
        d = pltpu.make_async_copy(acc.at[r], out_ref.at[pi_sm[row0 + r]],
                                  sem_out)
        d.start()
        dmas.append(d)
    for d in dmas:
        d.wait()


def _prep_call(fp2d, ti_f32, cr):
    return pl.pallas_call(
        _prep_body,
        out_shape=(
            jax.ShapeDtypeStruct((N, F), jnp.float32),
            jax.ShapeDtypeStruct((N, 1), jnp.float32),
            jax.ShapeDtypeStruct((N, 1), jnp.int32),
            jax.ShapeDtypeStruct((N, 1), jnp.int32),
        ),
    )(fp2d, ti_f32, cr)


def _combine_call(ci, pi, ws, cf, atlas3):
    grid_spec = pltpu.PrefetchScalarGridSpec(
        num_scalar_prefetch=2,
        grid=(NB,),
        in_specs=[
            pl.BlockSpec((BLK, F), lambda i, *_: (i, 0)),
            pl.BlockSpec((BLK, 1), lambda i, *_: (i, 0)),
            pl.BlockSpec(memory_space=pltpu.MemorySpace.HBM),
        ],
        out_specs=pl.BlockSpec(memory_space=pltpu.MemorySpace.HBM),
        scratch_shapes=[
            pltpu.VMEM((RB, F, D), jnp.float32),
            pltpu.VMEM((BLK, D), jnp.float32),
            pltpu.SemaphoreType.DMA((RB,)),
            pltpu.SemaphoreType.DMA,
        ],
    )
    return pl.pallas_call(
        _combine_body,
        grid_spec=grid_spec,
        out_shape=jax.ShapeDtypeStruct((N, D), jnp.float32),
    )(ci, pi, ws, cf, atlas3)


@jax.jit
def kernel(font_pred, char_labels, char_rec_vec, text_indexes,
           prerendered_alpha):
    del char_labels
    b, t, f = font_pred.shape
    fp2d = font_pred.reshape(b * t, f)
    ti_f32 = text_indexes.reshape(N, 1).astype(jnp.float32)
    ws, cf, ci, pi = _prep_call(fp2d, ti_f32, char_rec_vec)
    atlas3 = prerendered_alpha.reshape(F, C * D)
    out = _combine_call(ci.reshape(N), pi.reshape(N), ws, cf, atlas3)
    return out.reshape(N, 64, 64)


# speedup vs baseline: 1.4468x; 1.3269x over previous
import functools

import jax
import jax.numpy as jnp
from jax import lax
from jax.experimental import pallas as pl
from jax.experimental.pallas import tpu as pltpu

N = 2048
F = 100
C = 100
K = 20
D = 4096
BLK = 128
NB = N // BLK
CH = 512
NS = 4


def _prep_body(fp_ref, ti_ref, cr_ref, ws_ref, cf_ref, ci_ref, pi_ref):
    fp = fp_ref[...]
    ti = ti_ref[...]
    cr = cr_ref[...]

    lane_n = lax.broadcasted_iota(jnp.int32, (CH, N), 1).astype(jnp.float32)
    fpg_chunks = []
    for j in range(N // CH):
        tic = ti[CH * j:CH * (j + 1)]
        oh = (lane_n == tic).astype(jnp.float32)
        fpg_chunks.append(
            jnp.dot(oh, fp, preferred_element_type=jnp.float32,
                    precision=lax.Precision.HIGHEST))
    fpg = jnp.concatenate(fpg_chunks, axis=0)

    m = jnp.max(fpg, axis=1, keepdims=True)
    z = jnp.sum(jnp.exp(fpg - m), axis=1, keepdims=True)
    lane_f = lax.broadcasted_iota(jnp.int32, (N, F), 1).astype(jnp.float32)
    x = fpg
    wd = jnp.zeros((N, F), jnp.float32)
    for _ in range(K):
        vmax = jnp.max(x, axis=1, keepdims=True)
        fidx = jnp.min(jnp.where(x == vmax, lane_f, jnp.float32(1e9)),
                       axis=1, keepdims=True)
        sel = lane_f == fidx
        w = jnp.exp(vmax - m) / (z * 255.0)
        wd = wd + jnp.where(sel, w, 0.0)
        x = jnp.where(sel, jnp.float32(-1e30), x)

    cm = jnp.max(cr, axis=1, keepdims=True)
    cidx = jnp.min(jnp.where(cr == cm, lane_f, jnp.float32(1e9)),
                   axis=1, keepdims=True)

    onehot_c = (lane_f == cidx).astype(jnp.float32)
    cum = onehot_c
    s = 1
    while s < N:
        cum = cum + jnp.concatenate(
            [jnp.zeros((s, C), jnp.float32), cum[:N - s]], axis=0)
        s *= 2
    counts = cum[N - 1:N]
    inc = counts
    s = 1
    while s < C:
        inc = inc + jnp.concatenate(
            [jnp.zeros((1, s), jnp.float32), inc[:, :C - s]], axis=1)
        s *= 2
    offs = inc - counts
    rank = jnp.sum(onehot_c * cum, axis=1, keepdims=True) - 1.0
    base = jnp.sum(onehot_c * offs, axis=1, keepdims=True)
    pos = base + rank

    lane_ch = lax.broadcasted_iota(jnp.int32, (N, CH), 1).astype(jnp.float32)
    n_iota = lax.broadcasted_iota(jnp.int32, (N, 1), 0).astype(jnp.float32)
    dn = (((0,), (0,)), ((), ()))
    for j in range(N // CH):
        t = (pos == lane_ch + jnp.float32(CH * j)).astype(jnp.float32)
        ws_ref[CH * j:CH * (j + 1), :] = lax.dot_general(
            t, wd, dn, preferred_element_type=jnp.float32,
            precision=lax.Precision.HIGHEST)
        cf = lax.dot_general(t, cidx, dn, preferred_element_type=jnp.float32,
                             precision=lax.Precision.HIGHEST)
        pf = lax.dot_general(t, n_iota, dn, preferred_element_type=jnp.float32,
                             precision=lax.Precision.HIGHEST)
        cf_ref[CH * j:CH * (j + 1), :] = cf
        ci_ref[CH * j:CH * (j + 1), :] = cf.astype(jnp.int32)
        pi_ref[CH * j:CH * (j + 1), :] = pf.astype(jnp.int32)


def _combine_body(ci_sm, pi_sm, w_ref, cvec_ref, atlas_ref, out_ref,
                  g_buf, acc, sem_g, sem_out):
    i = pl.program_id(0)
    row0 = i * BLK
    c_lo = ci_sm[row0]
    c_hi = ci_sm[row0 + BLK - 1]
    num = c_hi - c_lo + 1

    def sub_copies(t, slot):
        c = c_lo + t
        bnd = (0, 32, 64, 96, F)
        return [
            pltpu.make_async_copy(
                atlas_ref.at[pl.ds(bnd[s], bnd[s + 1] - bnd[s]), c],
                g_buf.at[slot, pl.ds(bnd[s], bnd[s + 1] - bnd[s])],
                sem_g.at[slot, s])
            for s in range(NS)
        ]

    def issue(t, slot):
        for d in sub_copies(t, slot):
            d.start()

    issue(0, 0)
    acc[...] = jnp.zeros((BLK, D), jnp.float32)
    cvec = cvec_ref[...]
    w = w_ref[...]

    def body(t, carry):
        slot = lax.rem(t, 2)

        @pl.when(t + 1 < num)
        def _():
            issue(t + 1, 1 - slot)

        for d in sub_copies(t, slot):
            d.wait()
        mask = (cvec == (c_lo + t).astype(jnp.float32)).astype(jnp.float32)
        acc[...] = acc[...] + jnp.dot(w * mask, g_buf[slot],
                                      preferred_element_type=jnp.float32)
        return carry

    lax.fori_loop(0, num, body, 0)

    dmas = []
    for r in range(BLK):
        d = pltpu.make_async_copy(acc.at[r], out_ref.at[pi_sm[row0 + r]],
                                  sem_out)
        d.start()
        dmas.append(d)
    for d in dmas:
        d.wait()


def _prep_call(fp2d, ti_f32, cr):
    return pl.pallas_call(
        _prep_body,
        out_shape=(
            jax.ShapeDtypeStruct((N, F), jnp.float32),
            jax.ShapeDtypeStruct((N, 1), jnp.float32),
            jax.ShapeDtypeStruct((N, 1), jnp.int32),
            jax.ShapeDtypeStruct((N, 1), jnp.int32),
        ),
    )(fp2d, ti_f32, cr)


def _combine_call(ci, pi, ws, cf, atlas3):
    grid_spec = pltpu.PrefetchScalarGridSpec(
        num_scalar_prefetch=2,
        grid=(NB,),
        in_specs=[
            pl.BlockSpec((BLK, F), lambda i, *_: (i, 0)),
            pl.BlockSpec((BLK, 1), lambda i, *_: (i, 0)),
            pl.BlockSpec(memory_space=pltpu.MemorySpace.HBM),
        ],
        out_specs=pl.BlockSpec(memory_space=pltpu.MemorySpace.HBM),
        scratch_shapes=[
            pltpu.VMEM((2, F, D), jnp.float32),
            pltpu.VMEM((BLK, D), jnp.float32),
            pltpu.SemaphoreType.DMA((2, NS)),
            pltpu.SemaphoreType.DMA,
        ],
    )
    return pl.pallas_call(
        _combine_body,
        grid_spec=grid_spec,
        out_shape=jax.ShapeDtypeStruct((N, D), jnp.float32),
    )(ci, pi, ws, cf, atlas3)


@jax.jit
def kernel(font_pred, char_labels, char_rec_vec, text_indexes,
           prerendered_alpha):
    del char_labels
    b, t, f = font_pred.shape
    fp2d = font_pred.reshape(b * t, f)
    ti_f32 = text_indexes.reshape(N, 1).astype(jnp.float32)
    ws, cf, ci, pi = _prep_call(fp2d, ti_f32, char_rec_vec)
    atlas3 = prerendered_alpha.reshape(F, C, D)
    out = _combine_call(ci.reshape(N), pi.reshape(N), ws, cf, atlas3)
    return out.reshape(N, 64, 64)


# speedup vs baseline: 1.5475x; 1.0696x over previous
import functools

import jax
import jax.numpy as jnp
from jax import lax
from jax.experimental import pallas as pl
from jax.experimental.pallas import tpu as pltpu

N = 2048
F = 100
C = 100
K = 20
D = 4096
BLK = 128
NB = N // BLK
CH = 512
NS = 4


def _prep_body(fp_ref, ti_ref, cr_ref, ws_ref, cf_ref, ci_ref, pi_ref):
    fp = fp_ref[...]
    ti = ti_ref[...]
    cr = cr_ref[...]

    lane_n = lax.broadcasted_iota(jnp.int32, (CH, N), 1).astype(jnp.float32)
    fpg_chunks = []
    for j in range(N // CH):
        tic = ti[CH * j:CH * (j + 1)]
        oh = (lane_n == tic).astype(jnp.float32)
        fpg_chunks.append(
            jnp.dot(oh, fp, preferred_element_type=jnp.float32))
    fpg = jnp.concatenate(fpg_chunks, axis=0)

    m = jnp.max(fpg, axis=1, keepdims=True)
    z = jnp.sum(jnp.exp(fpg - m), axis=1, keepdims=True)
    lane_f = lax.broadcasted_iota(jnp.int32, (N, F), 1).astype(jnp.float32)
    x = fpg
    wd = jnp.zeros((N, F), jnp.float32)
    for _ in range(K):
        vmax = jnp.max(x, axis=1, keepdims=True)
        fidx = jnp.min(jnp.where(x == vmax, lane_f, jnp.float32(1e9)),
                       axis=1, keepdims=True)
        sel = lane_f == fidx
        w = jnp.exp(vmax - m) / (z * 255.0)
        wd = wd + jnp.where(sel, w, 0.0)
        x = jnp.where(sel, jnp.float32(-1e30), x)

    cm = jnp.max(cr, axis=1, keepdims=True)
    cidx = jnp.min(jnp.where(cr == cm, lane_f, jnp.float32(1e9)),
                   axis=1, keepdims=True)

    onehot_c = (lane_f == cidx).astype(jnp.float32)
    cum = onehot_c
    s = 1
    while s < N:
        cum = cum + jnp.concatenate(
            [jnp.zeros((s, C), jnp.float32), cum[:N - s]], axis=0)
        s *= 2
    counts = cum[N - 1:N]
    inc = counts
    s = 1
    while s < C:
        inc = inc + jnp.concatenate(
            [jnp.zeros((1, s), jnp.float32), inc[:, :C - s]], axis=1)
        s *= 2
    offs = inc - counts
    rank = jnp.sum(onehot_c * cum, axis=1, keepdims=True) - 1.0
    base = jnp.sum(onehot_c * offs, axis=1, keepdims=True)
    pos = base + rank

    lane_ch = lax.broadcasted_iota(jnp.int32, (N, CH), 1).astype(jnp.float32)
    n_iota = lax.broadcasted_iota(jnp.int32, (N, 1), 0).astype(jnp.float32)
    dn = (((0,), (0,)), ((), ()))
    for j in range(N // CH):
        t = (pos == lane_ch + jnp.float32(CH * j)).astype(jnp.float32)
        ws_ref[CH * j:CH * (j + 1), :] = lax.dot_general(
            t, wd, dn, preferred_element_type=jnp.float32)
        cf = lax.dot_general(t, cidx, dn, preferred_element_type=jnp.float32,
                             precision=lax.Precision.HIGHEST)
        pf = lax.dot_general(t, n_iota, dn, preferred_element_type=jnp.float32,
                             precision=lax.Precision.HIGHEST)
        cf_ref[CH * j:CH * (j + 1), :] = cf
        ci_ref[CH * j:CH * (j + 1), :] = cf.astype(jnp.int32)
        pi_ref[CH * j:CH * (j + 1), :] = pf.astype(jnp.int32)


def _combine_body(ci_sm, pi_sm, w_ref, cvec_ref, atlas_ref, out_ref,
                  g_buf, acc, sem_g, sem_out):
    i = pl.program_id(0)
    row0 = i * BLK
    c_lo = ci_sm[row0]
    c_hi = ci_sm[row0 + BLK - 1]
    num = c_hi - c_lo + 1
    prev_hi = ci_sm[jnp.maximum(row0 - 1, 0)]
    reuse = jnp.logical_and(i > 0, c_lo == prev_hi)

    def soff_step(b, s):
        lo_b = ci_sm[b * BLK]
        hi_p = ci_sm[b * BLK - 1]
        lo_p = ci_sm[(b - 1) * BLK]
        return jnp.where(lo_b == hi_p, lax.rem(hi_p - lo_p + s, 2), 0)

    soff = lax.fori_loop(1, i + 1, soff_step, 0)

    def slot_of(t):
        return lax.rem(t + soff, 2)

    def sub_copies(t, slot):
        c = c_lo + t
        bnd = (0, 32, 64, 96, F)
        return [
            pltpu.make_async_copy(
                atlas_ref.at[pl.ds(bnd[s], bnd[s + 1] - bnd[s]), c],
                g_buf.at[slot, pl.ds(bnd[s], bnd[s + 1] - bnd[s])],
                sem_g.at[slot, s])
            for s in range(NS)
        ]

    def issue(t, slot):
        for d in sub_copies(t, slot):
            d.start()

    @pl.when(jnp.logical_not(reuse))
    def _():
        issue(0, soff)

    acc[...] = jnp.zeros((BLK, D), jnp.float32)
    cvec = cvec_ref[...]
    w = w_ref[...]

    def body(t, carry):
        slot = slot_of(t)

        @pl.when(t + 1 < num)
        def _():
            issue(t + 1, 1 - slot)

        @pl.when(jnp.logical_or(t > 0, jnp.logical_not(reuse)))
        def _():
            for d in sub_copies(t, slot):
                d.wait()
        mask = (cvec == (c_lo + t).astype(jnp.float32)).astype(jnp.float32)
        acc[...] = acc[...] + jnp.dot(w * mask, g_buf[slot],
                                      preferred_element_type=jnp.float32)
        return carry

    lax.fori_loop(0, num, body, 0)

    dmas = []
    for r in range(BLK):
        d = pltpu.make_async_copy(acc.at[r], out_ref.at[pi_sm[row0 + r]],
                                  sem_out)
        d.start()
        dmas.append(d)
    for d in dmas:
        d.wait()


def _prep_call(fp2d, ti_f32, cr):
    return pl.pallas_call(
        _prep_body,
        out_shape=(
            jax.ShapeDtypeStruct((N, F), jnp.float32),
            jax.ShapeDtypeStruct((N, 1), jnp.float32),
            jax.ShapeDtypeStruct((N, 1), jnp.int32),
            jax.ShapeDtypeStruct((N, 1), jnp.int32),
        ),
    )(fp2d, ti_f32, cr)


def _combine_call(ci, pi, ws, cf, atlas3):
    grid_spec = pltpu.PrefetchScalarGridSpec(
        num_scalar_prefetch=2,
        grid=(NB,),
        in_specs=[
            pl.BlockSpec((BLK, F), lambda i, *_: (i, 0)),
            pl.BlockSpec((BLK, 1), lambda i, *_: (i, 0)),
            pl.BlockSpec(memory_space=pltpu.MemorySpace.HBM),
        ],
        out_specs=pl.BlockSpec(memory_space=pltpu.MemorySpace.HBM),
        scratch_shapes=[
            pltpu.VMEM((2, F, D), jnp.float32),
            pltpu.VMEM((BLK, D), jnp.float32),
            pltpu.SemaphoreType.DMA((2, NS)),
            pltpu.SemaphoreType.DMA,
        ],
    )
    return pl.pallas_call(
        _combine_body,
        grid_spec=grid_spec,
        out_shape=jax.ShapeDtypeStruct((N, D), jnp.float32),
    )(ci, pi, ws, cf, atlas3)


@jax.jit
def kernel(font_pred, char_labels, char_rec_vec, text_indexes,
           prerendered_alpha):
    del char_labels
    b, t, f = font_pred.shape
    fp2d = font_pred.reshape(b * t, f)
    ti_f32 = text_indexes.reshape(N, 1).astype(jnp.float32)
    ws, cf, ci, pi = _prep_call(fp2d, ti_f32, char_rec_vec)
    atlas3 = prerendered_alpha.reshape(F, C, D)
    out = _combine_call(ci.reshape(N), pi.reshape(N), ws, cf, atlas3)
    return out.reshape(N, 64, 64)
